# baseline (speedup 1.0000x reference)
import jax
import jax.numpy as jnp
from jax import lax
from jax.experimental import pallas as pl
from jax.experimental.pallas import tpu as pltpu, tpu_sc as plsc

VOCAB = 1000000
D = 64
B = 4096
S = 50
N = B * S

_info = plsc.get_sparse_core_info()
NC, NS = _info.num_cores, _info.num_subcores
NW = NC * NS
BLK = B // NW
L = 16
NG = BLK // L
DU = 8


def _body(xg_hbm, tab_hbm, out_hbm, xv, pi0, pi1, pr0, pr1, st0, st1,
          gs0, gs1, os0, os1):
    wid = lax.axis_index("s") * NC + lax.axis_index("c")
    pltpu.sync_copy(xg_hbm.at[wid], xv)
    iota = lax.iota(jnp.int32, L)

    def calc_pidx(s, pibuf):
        for g in range(NG):
            v = xv[s, pl.ds(g * L, L)]
            pibuf[pl.ds(g * L, L)] = lax.shift_right_logical(v, 1)

    def fire_gather(s, pibuf, prbuf, sem):
        calc_pidx(s, pibuf)
        pltpu.async_copy(tab_hbm.at[pibuf], prbuf, sem)

    def wait_gather(prbuf, sem):
        pltpu.make_async_copy(tab_hbm.at[pi0], prbuf, sem).wait()

    def assemble(s, prbuf, stbuf):
        for g in range(NG):
            v = xv[s, pl.ds(g * L, L)]
            rows = g * L + iota
            colbase = (v & 1) * D

            def dstep(dd, carry):
                for du in range(DU):
                    d = dd * DU + du
                    vals = plsc.load_gather(prbuf, [rows, colbase + d])
                    stbuf[d, pl.ds(g * L, L)] = vals
                return carry

            lax.fori_loop(0, D // DU, dstep, 0)

    def fire_out(s, stbuf, sem):
        pltpu.async_copy(stbuf, out_hbm.at[s, :, pl.ds(wid * BLK, BLK)], sem)

    def wait_out(s, stbuf, sem):
        pltpu.make_async_copy(
            stbuf, out_hbm.at[s, :, pl.ds(wid * BLK, BLK)], sem
        ).wait()

    fire_gather(0, pi0, pr0, gs0)
    fire_gather(1, pi1, pr1, gs1)
    wait_gather(pr0, gs0)
    assemble(0, pr0, st0)
    fire_out(0, st0, os0)
    wait_gather(pr1, gs1)
    assemble(1, pr1, st1)
    fire_out(1, st1, os1)

    def steady(m, carry):
        s0 = 2 * m
        fire_gather(s0, pi0, pr0, gs0)
        fire_gather(s0 + 1, pi1, pr1, gs1)
        wait_gather(pr0, gs0)
        wait_out(s0 - 2, st0, os0)
        assemble(s0, pr0, st0)
        fire_out(s0, st0, os0)
        wait_gather(pr1, gs1)
        wait_out(s0 - 1, st1, os1)
        assemble(s0 + 1, pr1, st1)
        fire_out(s0 + 1, st1, os1)
        return carry

    lax.fori_loop(1, S // 2, steady, 0)
    wait_out(S - 2, st0, os0)
    wait_out(S - 1, st1, os1)


def kernel(x, table):
    tab = table.reshape(VOCAB // 2, 2 * D)
    xg = x.T.reshape(S, NW, BLK).transpose(1, 0, 2)
    mesh = plsc.VectorSubcoreMesh(core_axis_name="c", subcore_axis_name="s")
    out = pl.kernel(
        _body,
        out_type=jax.ShapeDtypeStruct((S, D, B), jnp.float32),
        mesh=mesh,
        scratch_types=[
            pltpu.VMEM((S, BLK), jnp.int32),
            pltpu.VMEM((BLK,), jnp.int32),
            pltpu.VMEM((BLK,), jnp.int32),
            pltpu.VMEM((BLK, 2 * D), jnp.float32),
            pltpu.VMEM((BLK, 2 * D), jnp.float32),
            pltpu.VMEM((D, BLK), jnp.float32),
            pltpu.VMEM((D, BLK), jnp.float32),
            pltpu.SemaphoreType.DMA,
            pltpu.SemaphoreType.DMA,
            pltpu.SemaphoreType.DMA,
            pltpu.SemaphoreType.DMA,
        ],
        compiler_params=pltpu.CompilerParams(needs_layout_passes=False),
    )(xg, tab)
    return out.transpose(2, 0, 1)


# speedup vs baseline: 1.1772x; 1.1772x over previous
import jax
import jax.numpy as jnp
from jax import lax
from jax.experimental import pallas as pl
from jax.experimental.pallas import tpu as pltpu, tpu_sc as plsc

VOCAB = 1000000
D = 64
B = 4096
S = 50
N = B * S

_info = plsc.get_sparse_core_info()
NC, NS = _info.num_cores, _info.num_subcores
NW = NC * NS
BLK = B // NW
L = 16
NG = BLK // L
DU = 8


def _body(xg_hbm, tab_hbm, out_hbm, xv, pi0, pi1, pr0, pr1, st0, st1,
          gs0, gs1, os0, os1):
    wid = lax.axis_index("s") * NC + lax.axis_index("c")
    pltpu.sync_copy(xg_hbm.at[wid], xv)
    iota = lax.iota(jnp.int32, L)

    def calc_pidx(s, pibuf):
        for g in range(NG):
            v = xv[s, pl.ds(g * L, L)]
            pibuf[pl.ds(g * L, L)] = lax.shift_right_logical(v, 1)

    def fire_gather(s, pibuf, prbuf, sem):
        calc_pidx(s, pibuf)
        pltpu.async_copy(tab_hbm.at[pibuf], prbuf, sem)

    def wait_gather(prbuf, sem):
        pltpu.make_async_copy(tab_hbm.at[pi0], prbuf, sem).wait()

    def assemble(s, prbuf, stbuf):
        for g in range(NG):
            v = xv[s, pl.ds(g * L, L)]
            rows = g * L + iota
            colbase = (v & 1) * D

            @plsc.parallel_loop(0, D, step=1, unroll=DU)
            def dstep(d):
                vals = plsc.load_gather(prbuf, [rows, colbase + d])
                stbuf[d, pl.ds(g * L, L)] = vals

    def fire_out(s, stbuf, sem):
        pltpu.async_copy(stbuf, out_hbm.at[s, :, pl.ds(wid * BLK, BLK)], sem)

    def wait_out(s, stbuf, sem):
        pltpu.make_async_copy(
            stbuf, out_hbm.at[s, :, pl.ds(wid * BLK, BLK)], sem
        ).wait()

    fire_gather(0, pi0, pr0, gs0)
    fire_gather(1, pi1, pr1, gs1)
    wait_gather(pr0, gs0)
    assemble(0, pr0, st0)
    fire_out(0, st0, os0)
    wait_gather(pr1, gs1)
    assemble(1, pr1, st1)
    fire_out(1, st1, os1)

    def steady(m, carry):
        s0 = 2 * m
        fire_gather(s0, pi0, pr0, gs0)
        fire_gather(s0 + 1, pi1, pr1, gs1)
        wait_gather(pr0, gs0)
        wait_out(s0 - 2, st0, os0)
        assemble(s0, pr0, st0)
        fire_out(s0, st0, os0)
        wait_gather(pr1, gs1)
        wait_out(s0 - 1, st1, os1)
        assemble(s0 + 1, pr1, st1)
        fire_out(s0 + 1, st1, os1)
        return carry

    lax.fori_loop(1, S // 2, steady, 0)
    wait_out(S - 2, st0, os0)
    wait_out(S - 1, st1, os1)


def kernel(x, table):
    tab = table.reshape(VOCAB // 2, 2 * D)
    xg = x.T.reshape(S, NW, BLK).transpose(1, 0, 2)
    mesh = plsc.VectorSubcoreMesh(core_axis_name="c", subcore_axis_name="s")
    out = pl.kernel(
        _body,
        out_type=jax.ShapeDtypeStruct((S, D, B), jnp.float32),
        mesh=mesh,
        scratch_types=[
            pltpu.VMEM((S, BLK), jnp.int32),
            pltpu.VMEM((BLK,), jnp.int32),
            pltpu.VMEM((BLK,), jnp.int32),
            pltpu.VMEM((BLK, 2 * D), jnp.float32),
            pltpu.VMEM((BLK, 2 * D), jnp.float32),
            pltpu.VMEM((D, BLK), jnp.float32),
            pltpu.VMEM((D, BLK), jnp.float32),
            pltpu.SemaphoreType.DMA,
            pltpu.SemaphoreType.DMA,
            pltpu.SemaphoreType.DMA,
            pltpu.SemaphoreType.DMA,
        ],
        compiler_params=pltpu.CompilerParams(needs_layout_passes=False),
    )(xg, tab)
    return out.transpose(2, 0, 1)


# speedup vs baseline: 1.2441x; 1.0568x over previous
import jax
import jax.numpy as jnp
from jax import lax
from jax.experimental import pallas as pl
from jax.experimental.pallas import tpu as pltpu, tpu_sc as plsc

VOCAB = 1000000
D = 64
B = 4096
S = 50
N = B * S

_info = plsc.get_sparse_core_info()
NC, NS = _info.num_cores, _info.num_subcores
NW = NC * NS
BLK = B // NW
L = 16
NG = BLK // L
DU = 8
NPR = 4
PR_BYTES = BLK * 2 * D * 4
ST_BYTES = D * BLK * 4


def _body(xg_hbm, tab_hbm, out_hbm, xv, pix, pr0, pr1, pr2, pr3,
          st0, st1, gs0, gs1, gs2, gs3, os0, os1):
    wid = lax.axis_index("s") * NC + lax.axis_index("c")
    pltpu.sync_copy(xg_hbm.at[wid], xv)
    iota = lax.iota(jnp.int32, L)

    prs = (pr0, pr1, pr2, pr3)
    gss = (gs0, gs1, gs2, gs3)
    sts = (st0, st1)
    oss = (os0, os1)

    def fill(s, c):
        for g in range(NG):
            pix[s, pl.ds(g * L, L)] = lax.shift_right_logical(
                xv[s, pl.ds(g * L, L)], 1
            )
        return c

    lax.fori_loop(0, S, fill, 0)

    def fire_g(s, ph):
        pltpu.async_copy(tab_hbm.at[pix.at[s]], prs[ph], gss[ph])

    def fire_o(s, ph):
        pltpu.async_copy(
            sts[ph % 2], out_hbm.at[s, :, pl.ds(wid * BLK, BLK)], oss[ph % 2]
        )

    def assemble(s, ph):
        prbuf = prs[ph]
        stbuf = sts[ph % 2]
        for g in range(NG):
            v = xv[s, pl.ds(g * L, L)]
            rows = g * L + iota
            colbase = (v & 1) * D

            @plsc.parallel_loop(0, D, step=1, unroll=DU)
            def dstep(d):
                vals = plsc.load_gather(prbuf, [rows, colbase + d])
                stbuf[d, pl.ds(g * L, L)] = vals

    for ph in range(NPR):
        fire_g(ph, ph)

    for s0 in range(2):
        pltpu.make_async_copy(tab_hbm.at[pix.at[0]], prs[s0], gss[s0]).wait()
        assemble(s0, s0)
        fire_o(s0, s0)
        fire_g(s0 + NPR, s0)

    def steady(m, c):
        for ph2 in range(NPR):
            s = NPR * m + 2 + ph2
            ph = (ph2 + 2) % NPR
            pltpu.make_async_copy(tab_hbm.at[pix.at[0]], prs[ph], gss[ph]).wait()
            pltpu.make_async_copy(sts[ph % 2], out_hbm.at[0, :, pl.ds(wid * BLK, BLK)], oss[ph % 2]).wait()
            assemble(s, ph)
            fire_o(s, ph)
            fire_g(s + NPR, ph)
        return c

    lax.fori_loop(0, (S - 6) // NPR, steady, 0)

    for ph2 in range(NPR):
        s = S - NPR + ph2
        ph = (ph2 + 2) % NPR
        pltpu.make_async_copy(tab_hbm.at[pix.at[0]], prs[ph], gss[ph]).wait()
        pltpu.make_async_copy(sts[ph % 2], out_hbm.at[0, :, pl.ds(wid * BLK, BLK)], oss[ph % 2]).wait()
        assemble(s, ph)
        fire_o(s, ph)

    pltpu.make_async_copy(st0, out_hbm.at[0, :, pl.ds(wid * BLK, BLK)], os0).wait()
    pltpu.make_async_copy(st1, out_hbm.at[0, :, pl.ds(wid * BLK, BLK)], os1).wait()


def kernel(x, table):
    tab = table.reshape(VOCAB // 2, 2 * D)
    xg = x.T.reshape(S, NW, BLK).transpose(1, 0, 2)
    mesh = plsc.VectorSubcoreMesh(core_axis_name="c", subcore_axis_name="s")
    out = pl.kernel(
        _body,
        out_type=jax.ShapeDtypeStruct((S, D, B), jnp.float32),
        mesh=mesh,
        scratch_types=[
            pltpu.VMEM((S, BLK), jnp.int32),
            pltpu.VMEM((S + 2, BLK), jnp.int32),
            pltpu.VMEM((BLK, 2 * D), jnp.float32),
            pltpu.VMEM((BLK, 2 * D), jnp.float32),
            pltpu.VMEM((BLK, 2 * D), jnp.float32),
            pltpu.VMEM((BLK, 2 * D), jnp.float32),
            pltpu.VMEM((D, BLK), jnp.float32),
            pltpu.VMEM((D, BLK), jnp.float32),
            pltpu.SemaphoreType.DMA,
            pltpu.SemaphoreType.DMA,
            pltpu.SemaphoreType.DMA,
            pltpu.SemaphoreType.DMA,
            pltpu.SemaphoreType.DMA,
            pltpu.SemaphoreType.DMA,
        ],
        compiler_params=pltpu.CompilerParams(needs_layout_passes=False),
    )(xg, tab)
    return out.transpose(2, 0, 1)


# speedup vs baseline: 1.2549x; 1.0087x over previous
import jax
import jax.numpy as jnp
from jax import lax
from jax.experimental import pallas as pl
from jax.experimental.pallas import tpu as pltpu, tpu_sc as plsc

VOCAB = 1000000
D = 64
B = 4096
S = 50

_info = plsc.get_sparse_core_info()
NC, NS = _info.num_cores, _info.num_subcores
NW = NC * NS
BLK = B // NW
L = 16
NG = BLK // L
DU = 8
RING = 10


def _body(xg_hbm, tab_hbm, out_hbm, xv, pr, st, gsem, osem):
    wid = lax.axis_index("s") * NC + lax.axis_index("c")
    pltpu.sync_copy(xg_hbm.at[wid], xv)
    iota = lax.iota(jnp.int32, L)

    def fire_g(s, ph):
        pltpu.async_copy(tab_hbm.at[xv.at[s]], pr.at[ph], gsem.at[ph])

    def prime(s, c):
        fire_g(s, s)
        return c

    lax.fori_loop(0, RING, prime, 0)

    def step(s, c):
        ph = lax.rem(s, RING)
        pq = lax.rem(s, 2)
        prbuf = pr.at[ph]
        stbuf = st.at[pq]
        pltpu.make_async_copy(tab_hbm.at[xv.at[0]], prbuf, gsem.at[ph]).wait()

        @pl.when(s >= 2)
        def _():
            pltpu.make_async_copy(
                stbuf, out_hbm.at[0, :, wid], osem.at[pq]
            ).wait()

        for g in range(NG):
            rows = g * L + iota

            @plsc.parallel_loop(0, D, step=1, unroll=DU)
            def dstep(d):
                vals = plsc.load_gather(prbuf, [rows, iota * 0 + d])
                stbuf[d // 8, lax.rem(d, 8), pl.ds(g * L, L)] = vals

        pltpu.async_copy(stbuf, out_hbm.at[s, :, wid], osem.at[pq])

        @pl.when(s < S - RING)
        def _():
            fire_g(s + RING, ph)

        return c

    lax.fori_loop(0, S, step, 0)
    pltpu.make_async_copy(st.at[0], out_hbm.at[0, :, wid], osem.at[0]).wait()
    pltpu.make_async_copy(st.at[1], out_hbm.at[0, :, wid], osem.at[1]).wait()


def kernel(x, table):
    xg = x.T.reshape(S, NW, BLK).transpose(1, 0, 2)
    mesh = plsc.VectorSubcoreMesh(core_axis_name="c", subcore_axis_name="s")
    out5 = pl.kernel(
        _body,
        out_type=jax.ShapeDtypeStruct((S, D // 8, NW, 8, BLK), jnp.float32),
        mesh=mesh,
        scratch_types=[
            pltpu.VMEM((S, BLK), jnp.int32),
            pltpu.VMEM((RING, BLK, D), jnp.float32),
            pltpu.VMEM((2, D // 8, 8, BLK), jnp.float32),
            pltpu.SemaphoreType.DMA((RING,)),
            pltpu.SemaphoreType.DMA((2,)),
        ],
        compiler_params=pltpu.CompilerParams(
            use_tc_tiling_on_sc=False, needs_layout_passes=False
        ),
    )(xg, table)
    return out5.transpose(2, 4, 0, 1, 3).reshape(B, S, D)


# speedup vs baseline: 1.3551x; 1.0799x over previous
import jax
import jax.numpy as jnp
from jax import lax
from jax.experimental import pallas as pl
from jax.experimental.pallas import tpu as pltpu, tpu_sc as plsc

VOCAB = 1000000
D = 64
B = 4096
S = 50

_info = plsc.get_sparse_core_info()
NC, NS = _info.num_cores, _info.num_subcores
NW = NC * NS
BLK = B // NW
L = 16
NG = BLK // L
DU = 8
RING = 6


def _body(xg_hbm, tab_hbm, out_hbm, xv, pr, st, gsem, osem):
    wid = lax.axis_index("s") * NC + lax.axis_index("c")
    pltpu.sync_copy(xg_hbm.at[wid], xv)
    iota = lax.iota(jnp.int32, L)

    def fire_g(s, ph):
        pltpu.async_copy(tab_hbm.at[xv.at[s]], pr.at[ph], gsem.at[ph])

    def prime(s, c):
        fire_g(s, s)
        return c

    lax.fori_loop(0, RING, prime, 0)

    def step(s, c):
        ph = lax.rem(s, RING)
        pq = lax.rem(s, 2)
        prbuf = pr.at[ph]
        stbuf = st.at[pq]
        pltpu.make_async_copy(tab_hbm.at[xv.at[0]], prbuf, gsem.at[ph]).wait()

        @pl.when(s >= 2)
        def _():
            pltpu.make_async_copy(
                stbuf, out_hbm.at[0, :, wid], osem.at[pq]
            ).wait()

        for g in range(NG):
            rows = g * L + iota

            @plsc.parallel_loop(0, D, step=1, unroll=DU)
            def dstep(d):
                vals = plsc.load_gather(prbuf, [rows, iota * 0 + d])
                stbuf[d // 8, lax.rem(d, 8), pl.ds(g * L, L)] = vals

        pltpu.async_copy(stbuf, out_hbm.at[s, :, wid], osem.at[pq])

        @pl.when(s < S - RING)
        def _():
            fire_g(s + RING, ph)

        return c

    lax.fori_loop(0, S, step, 0)
    pltpu.make_async_copy(st.at[0], out_hbm.at[0, :, wid], osem.at[0]).wait()
    pltpu.make_async_copy(st.at[1], out_hbm.at[0, :, wid], osem.at[1]).wait()


def kernel(x, table):
    xg = x.T.reshape(S, NW, BLK).transpose(1, 0, 2)
    tab = jnp.pad(table, ((0, 0), (0, 2 * D - table.shape[1])))
    mesh = plsc.VectorSubcoreMesh(core_axis_name="c", subcore_axis_name="s")
    out5 = pl.kernel(
        _body,
        out_type=jax.ShapeDtypeStruct((S, D // 8, NW, 8, BLK), jnp.float32),
        mesh=mesh,
        scratch_types=[
            pltpu.VMEM((S, BLK), jnp.int32),
            pltpu.VMEM((RING, BLK, 2 * D), jnp.float32),
            pltpu.VMEM((2, D // 8, 8, BLK), jnp.float32),
            pltpu.SemaphoreType.DMA((RING,)),
            pltpu.SemaphoreType.DMA((2,)),
        ],
        compiler_params=pltpu.CompilerParams(needs_layout_passes=False),
    )(xg, tab)
    return out5.transpose(2, 4, 0, 1, 3).reshape(B, S, D)
